# baseline (speedup 1.0000x reference)
import functools

import jax
import jax.numpy as jnp
from jax import lax
from jax.experimental import pallas as pl
from jax.experimental.pallas import tpu as pltpu
from jax.experimental.pallas import tpu_sc as plsc

_VOC = 1000000
_D = 16
_B = 16384

_NC = 2
_NS = 16
_NW = _NC * _NS
_BPW = _B // _NW
_CHUNK = 128
_NCHUNK = _BPW // _CHUNK


def _sc_gather(table, qs32):
    mesh = plsc.VectorSubcoreMesh(core_axis_name="c", subcore_axis_name="s")

    @functools.partial(
        pl.kernel,
        mesh=mesh,
        out_type=jax.ShapeDtypeStruct((_B, _D), jnp.float32),
        scratch_types=[
            pltpu.VMEM((_BPW,), jnp.int32),
            pltpu.VMEM((_BPW, _D), jnp.float32),
            pltpu.SemaphoreType.DMA,
        ],
        compiler_params=pltpu.CompilerParams(use_tc_tiling_on_sc=False),
    )
    def gather_k(table_hbm, idx_hbm, out_hbm, idx_v, rows_v, sem):
        wid = lax.axis_index("s") * _NC + lax.axis_index("c")
        base = wid * _BPW
        pltpu.sync_copy(idx_hbm.at[pl.ds(base, _BPW)], idx_v)
        copies = []
        for k in range(_NCHUNK):
            copies.append(
                pltpu.async_copy(
                    table_hbm.at[idx_v.at[pl.ds(k * _CHUNK, _CHUNK)]],
                    rows_v.at[pl.ds(k * _CHUNK, _CHUNK), :],
                    sem,
                )
            )
        for cp in copies:
            cp.wait()
        pltpu.sync_copy(rows_v, out_hbm.at[pl.ds(base, _BPW)])

    return gather_k(table, qs32)


_JB = 1024
_IB = 2048


def _tc_correction(qs_col, idx_row, emb_update, vs_base):
    def body(qs_ref, idx_ref, emb_ref, base_ref, out_ref):
        @pl.when(pl.program_id(1) == 0)
        def _():
            out_ref[...] = base_ref[...]

        eq = (qs_ref[...] == idx_ref[...]).astype(jnp.float32)
        out_ref[...] += jnp.dot(eq, emb_ref[...],
                                preferred_element_type=jnp.float32,
                                precision=lax.Precision.HIGHEST)

    return pl.pallas_call(
        body,
        grid=(_B // _JB, _B // _IB),
        in_specs=[
            pl.BlockSpec((_JB, 1), lambda j, i: (j, 0)),
            pl.BlockSpec((1, _IB), lambda j, i: (0, i)),
            pl.BlockSpec((_IB, _D), lambda j, i: (i, 0)),
            pl.BlockSpec((_JB, _D), lambda j, i: (j, 0)),
        ],
        out_specs=pl.BlockSpec((_JB, _D), lambda j, i: (j, 0)),
        out_shape=jax.ShapeDtypeStruct((_B, _D), jnp.float32),
    )(qs_col, idx_row, emb_update, vs_base)


def kernel(indices, emb_update, qs, kernel):
    idx32 = indices.astype(jnp.int32)
    qs32 = qs.astype(jnp.int32)
    vs_base = _sc_gather(kernel, qs32)
    return _tc_correction(qs32.reshape(_B, 1), idx32.reshape(1, _B),
                          emb_update, vs_base)


# speedup vs baseline: 2.0225x; 2.0225x over previous
import functools

import jax
import jax.numpy as jnp
from jax import lax
from jax.experimental import pallas as pl
from jax.experimental.pallas import tpu as pltpu
from jax.experimental.pallas import tpu_sc as plsc

_VOC = 1000000
_D = 16
_B = 16384

_NC = 2
_NS = 16
_NW = _NC * _NS
_BPW = _B // _NW
_CHUNK = 128
_NCHUNK = _BPW // _CHUNK


def _sc_gather(table, qs32):
    mesh = plsc.VectorSubcoreMesh(core_axis_name="c", subcore_axis_name="s")

    @functools.partial(
        pl.kernel,
        mesh=mesh,
        out_type=jax.ShapeDtypeStruct((_B, _D), jnp.float32),
        scratch_types=[
            pltpu.VMEM((_BPW,), jnp.int32),
            pltpu.VMEM((_BPW, _D), jnp.float32),
            pltpu.SemaphoreType.DMA,
        ],
        compiler_params=pltpu.CompilerParams(use_tc_tiling_on_sc=False),
    )
    def gather_k(table_hbm, idx_hbm, out_hbm, idx_v, rows_v, sem):
        wid = lax.axis_index("s") * _NC + lax.axis_index("c")
        base = wid * _BPW
        pltpu.sync_copy(idx_hbm.at[pl.ds(base, _BPW)], idx_v)
        copies = []
        for k in range(_NCHUNK):
            copies.append(
                pltpu.async_copy(
                    table_hbm.at[idx_v.at[pl.ds(k * _CHUNK, _CHUNK)]],
                    rows_v.at[pl.ds(k * _CHUNK, _CHUNK), :],
                    sem,
                )
            )
        for cp in copies:
            cp.wait()
        pltpu.sync_copy(rows_v, out_hbm.at[pl.ds(base, _BPW)])

    return gather_k(table, qs32)


_JB = 1024
_IB = 2048


def _tc_correction(qs_col, idx_row, emb_update):
    def body(qs_ref, idx_ref, emb_ref, out_ref):
        eq = (qs_ref[...] == idx_ref[...]).astype(jnp.float32)
        prod = jnp.dot(eq, emb_ref[...], preferred_element_type=jnp.float32)

        @pl.when(pl.program_id(1) == 0)
        def _():
            out_ref[...] = prod

        @pl.when(pl.program_id(1) != 0)
        def _():
            out_ref[...] += prod

    return pl.pallas_call(
        body,
        grid=(_B // _JB, _B // _IB),
        in_specs=[
            pl.BlockSpec((_JB, 1), lambda j, i: (j, 0)),
            pl.BlockSpec((1, _IB), lambda j, i: (0, i)),
            pl.BlockSpec((_IB, _D), lambda j, i: (i, 0)),
        ],
        out_specs=pl.BlockSpec((_JB, _D), lambda j, i: (j, 0)),
        out_shape=jax.ShapeDtypeStruct((_B, _D), jnp.float32),
    )(qs_col, idx_row, emb_update)


def kernel(indices, emb_update, qs, kernel):
    idx32 = indices.astype(jnp.int32)
    qs32 = qs.astype(jnp.int32)
    vs_base = _sc_gather(kernel, qs32)
    corr = _tc_correction(qs32.reshape(_B, 1), idx32.reshape(1, _B),
                          emb_update)
    return vs_base + corr


# speedup vs baseline: 2.1260x; 1.0512x over previous
import functools

import jax
import jax.numpy as jnp
from jax import lax
from jax.experimental import pallas as pl
from jax.experimental.pallas import tpu as pltpu
from jax.experimental.pallas import tpu_sc as plsc

_VOC = 1000000
_D = 16
_B = 16384

_NC = 2
_NS = 16
_NW = _NC * _NS
_BPW = _B // _NW
_CHUNK = 128
_NCHUNK = _BPW // _CHUNK


def _sc_gather(table, qs32):
    mesh = plsc.VectorSubcoreMesh(core_axis_name="c", subcore_axis_name="s")

    @functools.partial(
        pl.kernel,
        mesh=mesh,
        out_type=jax.ShapeDtypeStruct((_B, _D), jnp.float32),
        scratch_types=[
            pltpu.VMEM((_BPW,), jnp.int32),
            pltpu.VMEM((_BPW, _D), jnp.float32),
            pltpu.SemaphoreType.DMA,
        ],
        compiler_params=pltpu.CompilerParams(use_tc_tiling_on_sc=False),
    )
    def gather_k(table_hbm, idx_hbm, out_hbm, idx_v, rows_v, sem):
        wid = lax.axis_index("s") * _NC + lax.axis_index("c")
        base = wid * _BPW
        pltpu.sync_copy(idx_hbm.at[pl.ds(base, _BPW)], idx_v)
        copies = []
        for k in range(_NCHUNK):
            copies.append(
                pltpu.async_copy(
                    table_hbm.at[idx_v.at[pl.ds(k * _CHUNK, _CHUNK)]],
                    rows_v.at[pl.ds(k * _CHUNK, _CHUNK), :],
                    sem,
                )
            )
        for cp in copies:
            cp.wait()
        pltpu.sync_copy(rows_v, out_hbm.at[pl.ds(base, _BPW)])

    return gather_k(table, qs32)


_JB = 4096
_IB = 8192


def _tc_correction(qs_col, idx_row, emb_update):
    def body(qs_ref, idx_ref, emb_ref, out_ref):
        eq = (qs_ref[...] == idx_ref[...]).astype(jnp.float32)
        prod = jnp.dot(eq, emb_ref[...], preferred_element_type=jnp.float32)

        @pl.when(pl.program_id(1) == 0)
        def _():
            out_ref[...] = prod

        @pl.when(pl.program_id(1) != 0)
        def _():
            out_ref[...] += prod

    return pl.pallas_call(
        body,
        grid=(_B // _JB, _B // _IB),
        in_specs=[
            pl.BlockSpec((_JB, 1), lambda j, i: (j, 0)),
            pl.BlockSpec((1, _IB), lambda j, i: (0, i)),
            pl.BlockSpec((_IB, _D), lambda j, i: (i, 0)),
        ],
        out_specs=pl.BlockSpec((_JB, _D), lambda j, i: (j, 0)),
        out_shape=jax.ShapeDtypeStruct((_B, _D), jnp.float32),
    )(qs_col, idx_row, emb_update)


def kernel(indices, emb_update, qs, kernel):
    idx32 = indices.astype(jnp.int32)
    qs32 = qs.astype(jnp.int32)
    vs_base = _sc_gather(kernel, qs32)
    corr = _tc_correction(qs32.reshape(_B, 1), idx32.reshape(1, _B),
                          emb_update)
    return vs_base + corr
